# baseline (speedup 1.0000x reference)
import jax
import jax.numpy as jnp
from jax import lax
from jax.experimental import pallas as pl
from jax.experimental.pallas import tpu as pltpu

_NUM_ACTIONS = 1000
_BATCH = 16384
_R = 1024
_GRID = _BATCH // _R


def _body(a_ref, q_ref, o_ref):
    q = q_ref[...]
    a = a_ref[...].reshape(_R, 1)
    iota = lax.broadcasted_iota(jnp.int32, (_R, _NUM_ACTIONS), 1)
    w = jnp.where(iota == a, q, 0.0)
    s = w[:, :128]
    for t in range(1, 7):
        s = s + w[:, t * 128 : (t + 1) * 128]
    tail = jnp.concatenate(
        [w[:, 896:1000], jnp.zeros((_R, 24), jnp.float32)], axis=1
    )
    s = s + tail
    out_lanes = jnp.sum(s.T, axis=0)
    o_ref[...] = out_lanes.reshape(1, 1, _R)


def kernel(actions, q_values):
    a3 = actions.astype(jnp.int32).reshape(_GRID, 1, _R)
    out = pl.pallas_call(
        _body,
        grid=(_GRID,),
        in_specs=[
            pl.BlockSpec((1, 1, _R), lambda i: (i, 0, 0)),
            pl.BlockSpec((_R, _NUM_ACTIONS), lambda i: (i, 0)),
        ],
        out_specs=pl.BlockSpec((1, 1, _R), lambda i: (i, 0, 0)),
        out_shape=jax.ShapeDtypeStruct((_GRID, 1, _R), jnp.float32),
        compiler_params=pltpu.CompilerParams(
            dimension_semantics=("arbitrary",),
            allow_input_fusion=[False, True],
        ),
    )(a3, q_values)
    return out.reshape(_BATCH)
